# baseline (speedup 1.0000x reference)
import functools

import jax
import jax.numpy as jnp
from jax import lax
from jax.experimental import pallas as pl
from jax.experimental.pallas import tpu as pltpu
from jax.experimental.pallas import tpu_sc as plsc

_NC = 2
_NS = 16


def kernel(inputs, dist_embedding):
    batch, seq = inputs.shape[0], inputs.shape[1]
    emb = dist_embedding.shape[1]
    total = seq * (seq + 1) // 2
    nrows = batch * total
    assert batch == 2 and seq % (2 * _NS) == 0

    starts = [k * seq - (k * (k - 1)) // 2 for k in range(seq)]
    ngroups = _NC * _NS // batch
    pairs_per_g = (seq // 2) // ngroups

    mesh = plsc.VectorSubcoreMesh(core_axis_name="c", subcore_axis_name="s")

    @functools.partial(
        pl.kernel,
        mesh=mesh,
        out_type=jax.ShapeDtypeStruct((nrows, emb), jnp.float32),
        scratch_types=[
            pltpu.VMEM_SHARED((seq, emb), jnp.float32),
            pltpu.SemaphoreType.DMA,
        ],
        compiler_params=pltpu.CompilerParams(use_tc_tiling_on_sc=False),
    )
    def _copy_kernel(table_hbm, out_hbm, spmem, sem):
        cid = lax.axis_index("c")
        sid = lax.axis_index("s")
        wid = cid * _NS + sid

        rows_per = seq // _NS
        pltpu.sync_copy(
            table_hbm.at[pl.ds(sid * rows_per, rows_per)],
            spmem.at[pl.ds(sid * rows_per, rows_per)],
        )
        plsc.subcore_barrier()

        b_off = (wid % batch) * total
        g = wid // batch
        for G in range(ngroups):
            @pl.when(g == G)
            def _(G=G):
                copies = []
                for t in range(pairs_per_g):
                    p = G + ngroups * t
                    for kk, L in ((p, seq - p), (seq - 1 - p, p + 1)):
                        copies.append(pltpu.async_copy(
                            spmem.at[pl.ds(0, L)],
                            out_hbm.at[pl.ds(b_off + starts[kk], L)],
                            sem,
                        ))
                for c in copies:
                    c.wait()

    out = _copy_kernel(dist_embedding)
    return out.reshape(batch, total, emb)
